# baseline (speedup 1.0000x reference)
import functools

import jax
import jax.numpy as jnp
from jax import lax
from jax.experimental import pallas as pl
from jax.experimental.pallas import tpu as pltpu
from jax.experimental.pallas import tpu_sc as plsc

H = 512
NSTEP, NSP = 10000, 10240
NANS, NAP = 1000, 1024
GR_S = 160
GR_A = 32
B = 32
ACCR = GR_S + 8


def _cap(e, ngroups):
    return ((e + B - 1) // B) * B + ngroups * B


def _make_agg(with_step):
    mesh = plsc.VectorSubcoreMesh(core_axis_name="c", subcore_axis_name="s")

    out_type = ([jax.ShapeDtypeStruct((NSP * H,), jnp.float32)] * 2
                if with_step else []) \
        + [jax.ShapeDtypeStruct((NAP * H,), jnp.float32),
           jax.ShapeDtypeStruct((NAP * H,), jnp.float32)]

    scratch = {
        "acc1": pltpu.VMEM((ACCR * H,), jnp.float32),
        "nb_v": pltpu.VMEM((2064,), jnp.int32),
        "sidx": pltpu.VMEM((B,), jnp.int32),
        "didx": pltpu.VMEM((B,), jnp.int32),
        "gbuf": pltpu.VMEM((B, H), jnp.float32),
        "sem": pltpu.SemaphoreType.DMA,
    }

    def _mo8(v):
        return pl.multiple_of(v, 8)

    def body(*refs, acc1, nb_v, sidx, didx, gbuf, sem):
        if with_step:
            (xs, xa, li_s, li_d, ls_s, ls_d, le_s, le_d, lc_s, lc_d, nb_hbm,
             z_hbm, agg_i, agg_sm, agg_e, agg_c) = refs
            types = ((0, li_s, li_d, xs, agg_i, GR_S, 2),
                     (1, ls_s, ls_d, xs, agg_sm, GR_S, 2),
                     (2, le_s, le_d, xa, agg_e, GR_A, 1),
                     (3, lc_s, lc_d, xs, agg_c, GR_A, 1))
        else:
            (xs, xa, le_s, le_d, lc_s, lc_d, nb_hbm, z_hbm,
             agg_e, agg_c) = refs
            types = ((2, le_s, le_d, xa, agg_e, GR_A, 1),
                     (3, lc_s, lc_d, xs, agg_c, GR_A, 1))
        sc = lax.axis_index("c")
        lid = lax.axis_index("s")
        wid = lid * 2 + sc
        del z_hbm
        pltpu.sync_copy(nb_hbm, nb_v)
        zv16 = jnp.zeros((16,), jnp.float32)
        for typ, ls, ld, x, agg, gsz, npass in types:
            def pass_body(p, carry0, typ=typ, ls=ls, ld=ld, x=x, agg=agg,
                          gsz=gsz):
                g = p * 32 + wid
                nzw = (gsz + 8) * H

                def zbody(t, carry):
                    acc1[pl.ds(t * 16, 16)] = zv16
                    return carry

                lax.fori_loop(0, nzw // 16, zbody, 0)
                hv = nb_v[pl.ds(_mo8((typ * 64 + g) * 8), 16)]
                sb = hv[0]
                nb = hv[1]

                def blk(i, carry):
                    off = _mo8((sb + i) * B)
                    pltpu.sync_copy(ls.at[pl.ds(off, B)], sidx)
                    pltpu.sync_copy(ld.at[pl.ds(off, B)], didx)
                    pltpu.async_copy(x.at[sidx], gbuf, sem).wait()

                    def sub(q, carry2):
                        dvec = didx[pl.ds(q * 16, 16)] * H
                        for l in range(16):
                            dbase = dvec[l]
                            for c in range(H // 16):
                                v = gbuf[q * 16 + l, pl.ds(c * 16, 16)]
                                plsc.addupdate(
                                    acc1.at[pl.ds(dbase + c * 16, 16)], v)
                        return carry2

                    lax.fori_loop(0, B // 16, sub, 0)
                    return carry

                lax.fori_loop(0, nb, blk, 0)
                pltpu.sync_copy(
                    acc1.at[pl.ds(0, gsz * H)],
                    agg.at[pl.ds(_mo8(g * gsz * H), gsz * H)])
                return carry0

            lax.fori_loop(0, npass, pass_body, 0)

    return pl.kernel(body, out_type=out_type, mesh=mesh,
                     scratch_types=scratch)


def _partition(src, dst, gsz, ngroups, cap):
    e = src.shape[0]
    key = (dst // gsz).astype(jnp.int32)
    perm = jnp.argsort(key, stable=True)
    ks = key[perm]
    starts = jnp.searchsorted(ks, jnp.arange(ngroups, dtype=jnp.int32)).astype(jnp.int32)
    rank = jnp.arange(e, dtype=jnp.int32) - starts[ks]
    cnt = jnp.zeros((ngroups,), jnp.int32).at[key].add(1)
    nblk = (cnt + B - 1) // B
    sblk = jnp.concatenate([jnp.zeros((1,), jnp.int32), jnp.cumsum(nblk)[:-1].astype(jnp.int32)])
    pos = sblk[ks] * B + rank
    lsrc = jnp.zeros((cap,), jnp.int32).at[pos].set(src[perm])
    ldst = jnp.full((cap,), gsz, jnp.int32).at[pos].set(dst[perm] - ks * gsz)
    hdr = jnp.zeros((64, 8), jnp.int32).at[:ngroups, 0].set(sblk).at[:ngroups, 1].set(nblk)
    return lsrc, ldst, hdr
def _deg(dst, npad):
    d = jnp.zeros((npad,), jnp.float32).at[dst].add(1.0)
    return jnp.stack([d, jnp.zeros_like(d)])[..., None]


def _linear_body(a_ref, w_ref, b_ref, o_ref):
    acc = jnp.dot(a_ref[...], w_ref[...], preferred_element_type=jnp.float32)
    o_ref[...] = acc + b_ref[...]


def tc_linear(a, w, b, bm):
    m, k = a.shape
    n = w.shape[1]
    return pl.pallas_call(
        _linear_body,
        grid=(m // bm,),
        in_specs=[
            pl.BlockSpec((bm, k), lambda i: (i, 0)),
            pl.BlockSpec((k, n), lambda i: (0, 0)),
            pl.BlockSpec((1, n), lambda i: (0, 0)),
        ],
        out_specs=pl.BlockSpec((bm, n), lambda i: (i, 0)),
        out_shape=jax.ShapeDtypeStruct((m, n), jnp.float32),
    )(a, w, b.reshape(1, n))


def _inv(deg_blk):
    return 1.0 / jnp.clip(deg_blk[0] + deg_blk[1], 1.0, None)


def _sage_body(a1_ref, d1_ref, a2_ref, d2_ref, x_ref, w_ref, b_ref, o_ref):
    m1 = a1_ref[...] * _inv(d1_ref[...])
    m2 = a2_ref[...] * _inv(d2_ref[...])
    acc = jnp.dot(m1, w_ref[0], preferred_element_type=jnp.float32)
    acc += jnp.dot(m2, w_ref[1], preferred_element_type=jnp.float32)
    acc += jnp.dot(x_ref[...], w_ref[2], preferred_element_type=jnp.float32)
    o_ref[...] = jnp.maximum(acc + b_ref[...], 0.0)


def tc_sage(a1, d1, a2, d2, x, w3, bias, bm):
    m = a1.shape[0]
    return pl.pallas_call(
        _sage_body,
        grid=(m // bm,),
        in_specs=[
            pl.BlockSpec((bm, H), lambda i: (i, 0)),
            pl.BlockSpec((2, bm, 1), lambda i: (0, i, 0)),
            pl.BlockSpec((bm, H), lambda i: (i, 0)),
            pl.BlockSpec((2, bm, 1), lambda i: (0, i, 0)),
            pl.BlockSpec((bm, H), lambda i: (i, 0)),
            pl.BlockSpec((3, H, H), lambda i: (0, 0, 0)),
            pl.BlockSpec((1, H), lambda i: (0, 0)),
        ],
        out_specs=pl.BlockSpec((bm, H), lambda i: (i, 0)),
        out_shape=jax.ShapeDtypeStruct((m, H), jnp.float32),
    )(a1, d1, a2, d2, x, w3, bias.reshape(1, H))


def _sage_pred_body(a1_ref, d1_ref, a2_ref, d2_ref, x_ref, w_ref, b_ref,
                    pw_ref, pb_ref, o_ref):
    m1 = a1_ref[...] * _inv(d1_ref[...])
    m2 = a2_ref[...] * _inv(d2_ref[...])
    acc = jnp.dot(m1, w_ref[0], preferred_element_type=jnp.float32)
    acc += jnp.dot(m2, w_ref[1], preferred_element_type=jnp.float32)
    acc += jnp.dot(x_ref[...], w_ref[2], preferred_element_type=jnp.float32)
    h = jnp.maximum(acc + b_ref[...], 0.0)
    logits = jnp.sum(h * pw_ref[...], axis=1) + pb_ref[0, 0]
    o_ref[...] = jnp.broadcast_to(jax.nn.sigmoid(logits)[:, None], o_ref.shape)


def tc_sage_pred(a1, d1, a2, d2, x, w3, bias, pred_w, pred_b, bm):
    m = a1.shape[0]
    return pl.pallas_call(
        _sage_pred_body,
        grid=(m // bm,),
        in_specs=[
            pl.BlockSpec((bm, H), lambda i: (i, 0)),
            pl.BlockSpec((2, bm, 1), lambda i: (0, i, 0)),
            pl.BlockSpec((bm, H), lambda i: (i, 0)),
            pl.BlockSpec((2, bm, 1), lambda i: (0, i, 0)),
            pl.BlockSpec((bm, H), lambda i: (i, 0)),
            pl.BlockSpec((3, H, H), lambda i: (0, 0, 0)),
            pl.BlockSpec((1, H), lambda i: (0, 0)),
            pl.BlockSpec((1, H), lambda i: (0, 0)),
            pl.BlockSpec((1, 1), lambda i: (0, 0), memory_space=pltpu.SMEM),
        ],
        out_specs=pl.BlockSpec((bm, 128), lambda i: (i, 0)),
        out_shape=jax.ShapeDtypeStruct((m, 128), jnp.float32),
    )(a1, d1, a2, d2, x, w3, bias.reshape(1, H),
      pred_w.reshape(1, H), pred_b.reshape(1, 1))[:, 0]


def kernel(step_x, ei_implies, ei_semantic, ei_equiv, ei_contrib,
           step_proj_W, step_proj_b, ans_emb, ans_proj_W, ans_proj_b,
           conv1_Wl, conv1_bl, conv1_Wr, conv2_Wl, conv2_bl, conv2_Wr,
           pred_W, pred_b):
    cap_s = _cap(ei_implies.shape[1], 64)
    cap_e = _cap(ei_equiv.shape[1], 32)
    cap_c = _cap(ei_contrib.shape[1], 32)

    step_xp = jnp.pad(step_x, ((0, NSP - NSTEP), (0, 0)))
    ans_p = jnp.pad(ans_emb, ((0, NAP - NANS), (0, 0)))
    x_step = tc_linear(step_xp, step_proj_W, step_proj_b, bm=1280)
    x_ans = tc_linear(ans_p, ans_proj_W, ans_proj_b, bm=1024)

    li_s, li_d, nb_i = _partition(ei_implies[0], ei_implies[1], GR_S, 64,
                                  cap_s)
    ls_s, ls_d, nb_sm = _partition(ei_semantic[0], ei_semantic[1], GR_S, 64,
                                   cap_s)
    le_s, le_d, nb_e = _partition(ei_equiv[0], ei_equiv[1], GR_A, 32, cap_e)
    lc_s, lc_d, nb_c = _partition(ei_contrib[0], ei_contrib[1], GR_A, 32,
                                  cap_c)
    nb_all = jnp.pad(jnp.stack([nb_i, nb_sm, nb_e, nb_c]).reshape(-1),
                     (0, 16))
    zeros32 = jnp.zeros((4096,), jnp.float32)

    deg_i = _deg(ei_implies[1], NSP)
    deg_s = _deg(ei_semantic[1], NSP)
    deg_e = _deg(ei_equiv[1], NAP)
    deg_c = _deg(ei_contrib[1], NAP)

    agg_full = _make_agg(True)
    agg_ans = _make_agg(False)

    w_step = [jnp.stack([cWl[0], cWl[1], cWr[0] + cWr[1]])
              for cWl, cWr in ((conv1_Wl, conv1_Wr), (conv2_Wl, conv2_Wr))]
    b_step = [conv1_bl[0] + conv1_bl[1], conv2_bl[0] + conv2_bl[1]]
    w_ans = [jnp.stack([cWl[2], cWl[3], cWr[2] + cWr[3]])
             for cWl, cWr in ((conv1_Wl, conv1_Wr), (conv2_Wl, conv2_Wr))]
    b_ans = [conv1_bl[2] + conv1_bl[3], conv2_bl[2] + conv2_bl[3]]

    for layer in range(6):
        p = layer % 2
        if layer < 5:
            agg_i, agg_sm, agg_e, agg_c = agg_full(
                x_step, x_ans, li_s, li_d, ls_s, ls_d, le_s, le_d, lc_s, lc_d,
                nb_all, zeros32)
            new_step = tc_sage(agg_i.reshape(NSP, H), deg_i,
                               agg_sm.reshape(NSP, H), deg_s, x_step,
                               w_step[p], b_step[p], bm=1280)
            x_ans = tc_sage(agg_e.reshape(NAP, H), deg_e,
                            agg_c.reshape(NAP, H), deg_c, x_ans,
                            w_ans[p], b_ans[p], bm=1024)
            x_step = new_step
        else:
            agg_e, agg_c = agg_ans(x_step, x_ans, le_s, le_d, lc_s, lc_d,
                                   nb_all, zeros32)
            return tc_sage_pred(agg_e.reshape(NAP, H), deg_e,
                                agg_c.reshape(NAP, H), deg_c, x_ans,
                                w_ans[p], b_ans[p], pred_W[:, 0], pred_b,
                                bm=1024)[:NANS]


# speedup vs baseline: 1.4405x; 1.4405x over previous
import functools

import jax
import jax.numpy as jnp
from jax import lax
from jax.experimental import pallas as pl
from jax.experimental.pallas import tpu as pltpu
from jax.experimental.pallas import tpu_sc as plsc

H = 512
NSTEP, NSP = 10000, 10240
NANS, NAP = 1000, 1024
GR_S = 160
GR_A = 32
B = 32
ACCR = GR_S + 8


def _nbt(e, ngroups):
    return (e + B - 1) // B + 2 * ngroups + 4


def _make_agg(with_step):
    mesh = plsc.VectorSubcoreMesh(core_axis_name="c", subcore_axis_name="s")

    out_type = ([jax.ShapeDtypeStruct((NSP * H,), jnp.float32)] * 2
                if with_step else []) \
        + [jax.ShapeDtypeStruct((NAP * H,), jnp.float32),
           jax.ShapeDtypeStruct((NAP * H,), jnp.float32)]

    scratch = {
        "acc1": pltpu.VMEM((ACCR * H,), jnp.float32),
        "nb_v": pltpu.VMEM((2064,), jnp.int32),
        "cidx0": pltpu.VMEM((2 * B,), jnp.int32),
        "cidx1": pltpu.VMEM((2 * B,), jnp.int32),
        "gbuf0": pltpu.VMEM((B, H), jnp.float32),
        "gbuf1": pltpu.VMEM((B, H), jnp.float32),
        "si0": pltpu.SemaphoreType.DMA,
        "si1": pltpu.SemaphoreType.DMA,
        "sg0": pltpu.SemaphoreType.DMA,
        "sg1": pltpu.SemaphoreType.DMA,
    }

    def _mo8(v):
        return pl.multiple_of(v, 8)

    def body(*refs, acc1, nb_v, cidx0, cidx1, gbuf0, gbuf1, si0, si1,
             sg0, sg1):
        if with_step:
            (xs, xa, cl_i, cl_s, cl_e, cl_c, nb_hbm,
             agg_i, agg_sm, agg_e, agg_c) = refs
            types = ((0, cl_i, xs, agg_i, GR_S, 2),
                     (1, cl_s, xs, agg_sm, GR_S, 2),
                     (2, cl_e, xa, agg_e, GR_A, 1),
                     (3, cl_c, xs, agg_c, GR_A, 1))
        else:
            (xs, xa, cl_e, cl_c, nb_hbm, agg_e, agg_c) = refs
            types = ((2, cl_e, xa, agg_e, GR_A, 1),
                     (3, cl_c, xs, agg_c, GR_A, 1))
        sc = lax.axis_index("c")
        lid = lax.axis_index("s")
        wid = lid * 2 + sc
        pltpu.sync_copy(nb_hbm, nb_v)
        zv16 = jnp.zeros((16,), jnp.float32)

        BANK = (GR_A + 8) * H

        def accum(cidx, gb, bbase):
            for q in range(B // 16):
                dvec = cidx[pl.ds(B + q * 16, 16)] * H + bbase
                dls = [dvec[l] for l in range(16)]

                def cloop(c, carry, q=q, dls=dls, gb=gb):
                    co = pl.multiple_of(c * 16, 16)
                    for l in range(16):
                        v = gb[q * 16 + l, pl.ds(co, 16)]
                        plsc.addupdate(acc1.at[pl.ds(dls[l] + co, 16)], v)
                    return carry

                lax.fori_loop(0, H // 16, cloop, 0)

        for typ, cl, x, agg, gsz, npass in types:
            nbanks = 4 if gsz == GR_A else 1

            def pass_body(p, carry0, typ=typ, cl=cl, x=x, agg=agg, gsz=gsz,
                          nbanks=nbanks):
                g = p * 32 + wid

                def zbody(t, carry):
                    acc1[pl.ds(t * 16, 16)] = zv16
                    return carry

                lax.fori_loop(0, ((gsz + 8) * H * nbanks) // 16, zbody, 0)
                hv = nb_v[pl.ds(_mo8((typ * 64 + g) * 8), 16)]
                sb = hv[0]
                nb = hv[1]

                def loff(i):
                    return _mo8((sb + i) * (2 * B))

                @pl.when(nb > 0)
                def _():
                    pltpu.sync_copy(cl.at[pl.ds(loff(0), 2 * B)], cidx0)
                    pltpu.async_copy(x.at[cidx0.at[pl.ds(0, B)]], gbuf0, sg0)
                    pltpu.async_copy(cl.at[pl.ds(loff(1), 2 * B)], cidx1, si1)

                def pair(h, carry, nbanks=nbanks):
                    last = h == (nb // 2) - 1
                    if nbanks == 4:
                        bb0 = ((2 * h) & 3) * BANK
                        bb1 = ((2 * h + 1) & 3) * BANK
                    else:
                        bb0 = bb1 = 0
                    pltpu.make_async_copy(cl.at[pl.ds(0, 2 * B)], cidx1,
                                          si1).wait()
                    pltpu.async_copy(x.at[cidx1.at[pl.ds(0, B)]], gbuf1, sg1)
                    pltpu.make_async_copy(x.at[pl.ds(0, B)], gbuf0, sg0).wait()
                    accum(cidx0, gbuf0, bb0)

                    @pl.when(jnp.logical_not(last))
                    def _():
                        pltpu.async_copy(cl.at[pl.ds(loff(2 * h + 2), 2 * B)],
                                         cidx0, si0)
                    pltpu.make_async_copy(x.at[pl.ds(0, B)], gbuf1, sg1).wait()
                    accum(cidx1, gbuf1, bb1)

                    @pl.when(jnp.logical_not(last))
                    def _():
                        pltpu.make_async_copy(cl.at[pl.ds(0, 2 * B)], cidx0,
                                              si0).wait()
                        pltpu.async_copy(x.at[cidx0.at[pl.ds(0, B)]], gbuf0,
                                         sg0)
                        pltpu.async_copy(cl.at[pl.ds(loff(2 * h + 3), 2 * B)],
                                         cidx1, si1)
                    return carry

                lax.fori_loop(0, nb // 2, pair, 0)
                if nbanks == 4:
                    def merge(i, carry):
                        o = pl.multiple_of(i * 16, 16)
                        acc1[pl.ds(o, 16)] = (
                            (acc1[pl.ds(o, 16)] + acc1[pl.ds(BANK + o, 16)])
                            + (acc1[pl.ds(2 * BANK + o, 16)]
                               + acc1[pl.ds(3 * BANK + o, 16)]))
                        return carry

                    lax.fori_loop(0, (gsz * H) // 16, merge, 0)
                pltpu.sync_copy(
                    acc1.at[pl.ds(0, gsz * H)],
                    agg.at[pl.ds(_mo8(g * gsz * H), gsz * H)])
                return carry0

            lax.fori_loop(0, npass, pass_body, 0)

    return pl.kernel(body, out_type=out_type, mesh=mesh,
                     scratch_types=scratch)


def _partition(src, dst, gsz, ngroups, nblk_tot):
    e = src.shape[0]
    key = (dst // gsz).astype(jnp.int32)
    perm = jnp.argsort(key, stable=True)
    ks = key[perm]
    starts = jnp.searchsorted(ks, jnp.arange(ngroups, dtype=jnp.int32)).astype(jnp.int32)
    rank = jnp.arange(e, dtype=jnp.int32) - starts[ks]
    cnt = jnp.zeros((ngroups,), jnp.int32).at[key].add(1)
    nblk = (cnt + B - 1) // B
    nblk = nblk + (nblk & 1)
    sblk = jnp.concatenate([jnp.zeros((1,), jnp.int32),
                            jnp.cumsum(nblk)[:-1].astype(jnp.int32)])
    blk = sblk[ks] + rank // B
    lane = rank % B
    cpos = blk * (2 * B) + lane
    init = jnp.tile(jnp.concatenate([jnp.zeros((B,), jnp.int32),
                                     jnp.full((B,), gsz, jnp.int32)]),
                    nblk_tot)
    clist = init.at[cpos].set(src[perm]).at[cpos + B].set(
        dst[perm] - ks * gsz)
    hdr = jnp.zeros((64, 8), jnp.int32).at[:ngroups, 0].set(sblk)\
        .at[:ngroups, 1].set(nblk)
    return clist, hdr


def _deg(dst, npad):
    d = jnp.zeros((npad,), jnp.float32).at[dst].add(1.0)
    return jnp.stack([d, jnp.zeros_like(d)])[..., None]


def _linear_body(a_ref, w_ref, b_ref, o_ref):
    acc = jnp.dot(a_ref[...], w_ref[...], preferred_element_type=jnp.float32)
    o_ref[...] = acc + b_ref[...]


def tc_linear(a, w, b, bm):
    m, k = a.shape
    n = w.shape[1]
    return pl.pallas_call(
        _linear_body,
        grid=(m // bm,),
        in_specs=[
            pl.BlockSpec((bm, k), lambda i: (i, 0)),
            pl.BlockSpec((k, n), lambda i: (0, 0)),
            pl.BlockSpec((1, n), lambda i: (0, 0)),
        ],
        out_specs=pl.BlockSpec((bm, n), lambda i: (i, 0)),
        out_shape=jax.ShapeDtypeStruct((m, n), jnp.float32),
    )(a, w, b.reshape(1, n))


def _cnt(deg_blk):
    return jnp.clip(deg_blk[0] + deg_blk[1], 1.0, None)


def _sage_body(a1_ref, d1_ref, a2_ref, d2_ref, x_ref, w_ref, b_ref, o_ref):
    m1 = a1_ref[...] / _cnt(d1_ref[...])
    m2 = a2_ref[...] / _cnt(d2_ref[...])
    x = x_ref[...]
    s0 = (jnp.dot(m1, w_ref[0], preferred_element_type=jnp.float32)
          + b_ref[0]) + jnp.dot(x, w_ref[2],
                                preferred_element_type=jnp.float32)
    s1 = (jnp.dot(m2, w_ref[1], preferred_element_type=jnp.float32)
          + b_ref[1]) + jnp.dot(x, w_ref[3],
                                preferred_element_type=jnp.float32)
    o_ref[...] = jnp.maximum(s0 + s1, 0.0)


def tc_sage(a1, d1, a2, d2, x, w3, bias, bm):
    m = a1.shape[0]
    return pl.pallas_call(
        _sage_body,
        grid=(m // bm,),
        in_specs=[
            pl.BlockSpec((bm, H), lambda i: (i, 0)),
            pl.BlockSpec((2, bm, 1), lambda i: (0, i, 0)),
            pl.BlockSpec((bm, H), lambda i: (i, 0)),
            pl.BlockSpec((2, bm, 1), lambda i: (0, i, 0)),
            pl.BlockSpec((bm, H), lambda i: (i, 0)),
            pl.BlockSpec((4, H, H), lambda i: (0, 0, 0)),
            pl.BlockSpec((2, H), lambda i: (0, 0)),
        ],
        out_specs=pl.BlockSpec((bm, H), lambda i: (i, 0)),
        out_shape=jax.ShapeDtypeStruct((m, H), jnp.float32),
    )(a1, d1, a2, d2, x, w3, bias)


def _sage_pred_body(a1_ref, d1_ref, a2_ref, d2_ref, x_ref, w_ref, b_ref,
                    pw_ref, pb_ref, o_ref):
    m1 = a1_ref[...] / _cnt(d1_ref[...])
    m2 = a2_ref[...] / _cnt(d2_ref[...])
    x = x_ref[...]
    s0 = (jnp.dot(m1, w_ref[0], preferred_element_type=jnp.float32)
          + b_ref[0]) + jnp.dot(x, w_ref[2],
                                preferred_element_type=jnp.float32)
    s1 = (jnp.dot(m2, w_ref[1], preferred_element_type=jnp.float32)
          + b_ref[1]) + jnp.dot(x, w_ref[3],
                                preferred_element_type=jnp.float32)
    h = jnp.maximum(s0 + s1, 0.0)
    logits = jnp.dot(h, pw_ref[...],
                     preferred_element_type=jnp.float32) + pb_ref[0, 0]
    o_ref[...] = jax.nn.sigmoid(logits)


def tc_sage_pred(a1, d1, a2, d2, x, w3, bias, pred_w, pred_b, bm):
    m = a1.shape[0]
    return pl.pallas_call(
        _sage_pred_body,
        grid=(m // bm,),
        in_specs=[
            pl.BlockSpec((bm, H), lambda i: (i, 0)),
            pl.BlockSpec((2, bm, 1), lambda i: (0, i, 0)),
            pl.BlockSpec((bm, H), lambda i: (i, 0)),
            pl.BlockSpec((2, bm, 1), lambda i: (0, i, 0)),
            pl.BlockSpec((bm, H), lambda i: (i, 0)),
            pl.BlockSpec((4, H, H), lambda i: (0, 0, 0)),
            pl.BlockSpec((2, H), lambda i: (0, 0)),
            pl.BlockSpec((H, 128), lambda i: (0, 0)),
            pl.BlockSpec((1, 1), lambda i: (0, 0), memory_space=pltpu.SMEM),
        ],
        out_specs=pl.BlockSpec((bm, 128), lambda i: (i, 0)),
        out_shape=jax.ShapeDtypeStruct((m, 128), jnp.float32),
    )(a1, d1, a2, d2, x, w3, bias,
      jnp.pad(pred_w[:, None], ((0, 0), (0, 127))),
      pred_b.reshape(1, 1))[:, 0]


def kernel(step_x, ei_implies, ei_semantic, ei_equiv, ei_contrib,
           step_proj_W, step_proj_b, ans_emb, ans_proj_W, ans_proj_b,
           conv1_Wl, conv1_bl, conv1_Wr, conv2_Wl, conv2_bl, conv2_Wr,
           pred_W, pred_b):
    nbt_s = _nbt(ei_implies.shape[1], 64)
    nbt_e = _nbt(ei_equiv.shape[1], 32)
    nbt_c = _nbt(ei_contrib.shape[1], 32)

    step_xp = jnp.pad(step_x, ((0, NSP - NSTEP), (0, 0)))
    ans_p = jnp.pad(ans_emb, ((0, NAP - NANS), (0, 0)))
    x_step = tc_linear(step_xp, step_proj_W, step_proj_b, bm=1280)
    x_ans = tc_linear(ans_p, ans_proj_W, ans_proj_b, bm=1024)

    cl_i, nb_i = _partition(ei_implies[0], ei_implies[1], GR_S, 64, nbt_s)
    cl_s, nb_sm = _partition(ei_semantic[0], ei_semantic[1], GR_S, 64, nbt_s)
    cl_e, nb_e = _partition(ei_equiv[0], ei_equiv[1], GR_A, 32, nbt_e)
    cl_c, nb_c = _partition(ei_contrib[0], ei_contrib[1], GR_A, 32, nbt_c)
    nb_all = jnp.pad(jnp.stack([nb_i, nb_sm, nb_e, nb_c]).reshape(-1),
                     (0, 16))

    deg_i = _deg(ei_implies[1], NSP)
    deg_s = _deg(ei_semantic[1], NSP)
    deg_e = _deg(ei_equiv[1], NAP)
    deg_c = _deg(ei_contrib[1], NAP)

    agg_full = _make_agg(True)
    agg_ans = _make_agg(False)

    w_step = [jnp.stack([cWl[0], cWl[1], cWr[0], cWr[1]])
              for cWl, cWr in ((conv1_Wl, conv1_Wr), (conv2_Wl, conv2_Wr))]
    b_step = [conv1_bl[0:2], conv2_bl[0:2]]
    w_ans = [jnp.stack([cWl[2], cWl[3], cWr[2], cWr[3]])
             for cWl, cWr in ((conv1_Wl, conv1_Wr), (conv2_Wl, conv2_Wr))]
    b_ans = [conv1_bl[2:4], conv2_bl[2:4]]

    for layer in range(6):
        p = layer % 2
        if layer < 5:
            agg_i, agg_sm, agg_e, agg_c = agg_full(
                x_step, x_ans, cl_i, cl_s, cl_e, cl_c, nb_all)
            new_step = tc_sage(agg_i.reshape(NSP, H), deg_i,
                               agg_sm.reshape(NSP, H), deg_s, x_step,
                               w_step[p], b_step[p], bm=1280)
            x_ans = tc_sage(agg_e.reshape(NAP, H), deg_e,
                            agg_c.reshape(NAP, H), deg_c, x_ans,
                            w_ans[p], b_ans[p], bm=1024)
            x_step = new_step
        else:
            agg_e, agg_c = agg_ans(x_step, x_ans, cl_e, cl_c, nb_all)
            return tc_sage_pred(agg_e.reshape(NAP, H), deg_e,
                                agg_c.reshape(NAP, H), deg_c, x_ans,
                                w_ans[p], b_ans[p], pred_W[:, 0], pred_b,
                                bm=1024)[:NANS]
